# Initial kernel scaffold; baseline (speedup 1.0000x reference)
import jax
import jax.numpy as jnp
from jax.experimental import pallas as pl


def kernel(x, edge_index, W1, b1, W2, b2, W3, b3, Wr, br):
    raise NotImplementedError("write your pallas kernel here")


# speedup vs baseline: 88.7000x; 88.7000x over previous
import functools

import jax
import jax.numpy as jnp
from jax import lax
from jax.experimental import pallas as pl
from jax.experimental.pallas import tpu as pltpu
from jax.experimental.pallas import tpu_sc as plsc

_N = 100000
_E = 6400000
_NPAD = 100352
_SLICE = _NPAD // 16
_LANE = 128
_WINDOWS = 50176
_EPAD = _WINDOWS * _LANE
_WTILE = _WINDOWS // 32
_KW = 16
_CHUNKS = _WTILE // _KW
_BT = 2048
_GRID = _NPAD // _BT

_MESH = dict(core_axis_name="c", subcore_axis_name="s")


def _deg_call(dst2):
  out_type = [jax.ShapeDtypeStruct((_NPAD,), jnp.float32) for _ in range(2)]
  scratch = [
      pltpu.VMEM_SHARED((_NPAD,), jnp.float32),
      pltpu.VMEM((2, _KW, _LANE), jnp.int32),
      pltpu.VMEM((_LANE,), jnp.float32),
      pltpu.VMEM((_SLICE,), jnp.float32),
      pltpu.SemaphoreType.DMA,
      pltpu.SemaphoreType.DMA,
  ]

  def body(dst_hbm, out0, out1, dsh, dbuf, ones, zbuf, stage_sem, ssem):
    c = lax.axis_index("c")
    s = lax.axis_index("s")
    wid = s * 2 + c
    r0 = s * _SLICE

    z16 = jnp.zeros((16,), jnp.float32)
    o16 = jnp.ones((16,), jnp.float32)

    def zb(i, carry):
      zbuf[pl.ds(i * 16, 16)] = z16
      return carry
    lax.fori_loop(0, _SLICE // 16, zb, 0)
    for i in range(_LANE // 16):
      ones[pl.ds(i * 16, 16)] = o16
    pltpu.sync_copy(zbuf, dsh.at[pl.ds(r0, _SLICE)])
    plsc.subcore_barrier()

    w_base = wid * _WTILE
    pltpu.async_copy(dst_hbm.at[pl.ds(w_base, _KW), :], dbuf.at[0], stage_sem)

    def chunk(t, carry):
      b = lax.rem(t, 2)
      w0 = w_base + t * _KW
      pltpu.make_async_copy(
          dst_hbm.at[pl.ds(w0, _KW), :], dbuf.at[b], stage_sem).wait()

      @pl.when(t < _CHUNKS - 1)
      def _prefetch():
        pltpu.async_copy(
            dst_hbm.at[pl.ds(w0 + _KW, _KW), :], dbuf.at[1 - b], stage_sem)

      sd = []
      for j in range(_KW):
        sd.append(
            pltpu.async_copy(ones, dsh.at[dbuf.at[b, j]], ssem, add=True))
      for d in sd:
        d.wait()
      return carry

    lax.fori_loop(0, _CHUNKS, chunk, 0)
    plsc.subcore_barrier()

    @pl.when(c == 0)
    def _w0():
      pltpu.sync_copy(dsh.at[pl.ds(r0, _SLICE)], out0.at[pl.ds(r0, _SLICE)])

    @pl.when(c == 1)
    def _w1():
      pltpu.sync_copy(dsh.at[pl.ds(r0, _SLICE)], out1.at[pl.ds(r0, _SLICE)])

  fn = pl.kernel(
      body,
      out_type=out_type,
      mesh=plsc.VectorSubcoreMesh(**_MESH),
      scratch_types=scratch,
  )
  return fn(dst2)


def _agg_call(F, src2, dst2, ycols):
  out_type = [jax.ShapeDtypeStruct((_NPAD,), jnp.float32) for _ in range(2 * F)]
  scratch = (
      [pltpu.VMEM_SHARED((_NPAD,), jnp.float32) for _ in range(2 * F)]
      + [pltpu.VMEM((2, _KW, _LANE), jnp.int32) for _ in range(2)]
      + [pltpu.VMEM((_KW, _LANE), jnp.float32) for _ in range(F)]
      + [pltpu.VMEM((_SLICE,), jnp.float32)]
      + [pltpu.SemaphoreType.DMA for _ in range(3)]
  )

  def body(*refs):
    src_hbm, dst_hbm = refs[0], refs[1]
    yin = refs[2:2 + F]
    zout = refs[2 + F:2 + 3 * F]
    p = 2 + 3 * F
    ysh = refs[p:p + F]; p += F
    zsh = refs[p:p + F]; p += F
    sbuf, dbuf = refs[p], refs[p + 1]; p += 2
    grows = refs[p:p + F]; p += F
    zbuf = refs[p]; p += 1
    stage_sem, gsem, ssem = refs[p], refs[p + 1], refs[p + 2]

    c = lax.axis_index("c")
    s = lax.axis_index("s")
    wid = s * 2 + c
    r0 = s * _SLICE

    z16 = jnp.zeros((16,), jnp.float32)

    def zb(i, carry):
      zbuf[pl.ds(i * 16, 16)] = z16
      return carry
    lax.fori_loop(0, _SLICE // 16, zb, 0)

    for f in range(F):
      pltpu.sync_copy(yin[f].at[pl.ds(r0, _SLICE)],
                      ysh[f].at[pl.ds(r0, _SLICE)])
      pltpu.sync_copy(zbuf, zsh[f].at[pl.ds(r0, _SLICE)])
    plsc.subcore_barrier()

    w_base = wid * _WTILE
    pltpu.async_copy(src_hbm.at[pl.ds(w_base, _KW), :], sbuf.at[0], stage_sem)
    pltpu.async_copy(dst_hbm.at[pl.ds(w_base, _KW), :], dbuf.at[0], stage_sem)

    def chunk(t, carry):
      b = lax.rem(t, 2)
      w0 = w_base + t * _KW
      pltpu.make_async_copy(
          src_hbm.at[pl.ds(w0, _KW), :], sbuf.at[b], stage_sem).wait()
      pltpu.make_async_copy(
          dst_hbm.at[pl.ds(w0, _KW), :], dbuf.at[b], stage_sem).wait()

      @pl.when(t < _CHUNKS - 1)
      def _prefetch():
        nb = 1 - b
        pltpu.async_copy(
            src_hbm.at[pl.ds(w0 + _KW, _KW), :], sbuf.at[nb], stage_sem)
        pltpu.async_copy(
            dst_hbm.at[pl.ds(w0 + _KW, _KW), :], dbuf.at[nb], stage_sem)

      gd = []
      for j in range(_KW):
        for f in range(F):
          gd.append(pltpu.async_copy(
              ysh[f].at[sbuf.at[b, j]], grows[f].at[j], gsem))
      for d in gd:
        d.wait()
      sd = []
      for j in range(_KW):
        for f in range(F):
          sd.append(pltpu.async_copy(
              grows[f].at[j], zsh[f].at[dbuf.at[b, j]], ssem, add=True))
      for d in sd:
        d.wait()
      return carry

    lax.fori_loop(0, _CHUNKS, chunk, 0)
    plsc.subcore_barrier()

    for f in range(F):
      @pl.when(c == 0)
      def _w0(f=f):
        pltpu.sync_copy(zsh[f].at[pl.ds(r0, _SLICE)],
                        zout[f].at[pl.ds(r0, _SLICE)])

      @pl.when(c == 1)
      def _w1(f=f):
        pltpu.sync_copy(zsh[f].at[pl.ds(r0, _SLICE)],
                        zout[F + f].at[pl.ds(r0, _SLICE)])

  fn = pl.kernel(
      body,
      out_type=out_type,
      mesh=plsc.VectorSubcoreMesh(**_MESH),
      scratch_types=scratch,
  )
  return fn(src2, dst2, *ycols)


_BS = pl.BlockSpec((_BT,), lambda i: (i,))
_SM = pl.BlockSpec(memory_space=pltpu.SMEM)
_F32 = functools.partial(jax.ShapeDtypeStruct, dtype=jnp.float32)


def _t1_call(x0, x1, dp0, dp1):
  def body(x0r, x1r, d0r, d1r, dinvr, y00r, y01r):
    dinv = lax.rsqrt(d0r[...] + d1r[...] + 1.0)
    dinvr[...] = dinv
    y00r[...] = x0r[...] * dinv
    y01r[...] = x1r[...] * dinv

  return pl.pallas_call(
      body,
      grid=(_GRID,),
      in_specs=[_BS] * 4,
      out_specs=[_BS] * 3,
      out_shape=[_F32((_NPAD,))] * 3,
  )(x0, x1, dp0, dp1)


def _t2_call(z0, y0, dinv, W1, b1, W2):
  def body(za0, za1, zb0, zb1, y00, y01, dinvr, W1r, b1r, W2r, *outs):
    dinv = dinvr[...]
    ax = [dinv * (za0[...] + zb0[...] + y00[...]),
          dinv * (za1[...] + zb1[...] + y01[...])]
    h = []
    for j in range(4):
      g = ax[0] * W1r[0, j] + ax[1] * W1r[1, j] + b1r[j]
      h.append(jnp.tanh(g))
    for j in range(4):
      acc = h[0] * W2r[0, j]
      for i in range(1, 4):
        acc = acc + h[i] * W2r[i, j]
      outs[j][...] = dinv * acc

  return pl.pallas_call(
      body,
      grid=(_GRID,),
      in_specs=[_BS] * 7 + [_SM] * 3,
      out_specs=[_BS] * 4,
      out_shape=[_F32((_NPAD,))] * 4,
  )(z0[0], z0[1], z0[2], z0[3], y0[0], y0[1], dinv, W1, b1, W2)


def _t3_call(z1, y1, dinv, b2, W3):
  def body(za0, za1, za2, za3, zb0, zb1, zb2, zb3,
           y10, y11, y12, y13, dinvr, b2r, W3r, o0, o1):
    dinv = dinvr[...]
    za = [za0, za1, za2, za3]
    zb = [zb0, zb1, zb2, zb3]
    yy = [y10, y11, y12, y13]
    h = []
    for j in range(4):
      h.append(jnp.tanh(dinv * (za[j][...] + zb[j][...] + yy[j][...])
                        + b2r[j]))
    for k, o in enumerate((o0, o1)):
      acc = h[0] * W3r[0, k]
      for i in range(1, 4):
        acc = acc + h[i] * W3r[i, k]
      o[...] = dinv * acc

  return pl.pallas_call(
      body,
      grid=(_GRID,),
      in_specs=[_BS] * 13 + [_SM] * 2,
      out_specs=[_BS] * 2,
      out_shape=[_F32((_NPAD,))] * 2,
  )(*z1, *y1, dinv, b2, W3)


def _t4_call(z2, y2, dinv, b3, Wr):
  def body(za0, za1, zb0, zb1, y20, y21, dinvr, b3r, Wrr, outr):
    i = pl.program_id(0)
    dinv = dinvr[...]
    h0 = jnp.tanh(dinv * (za0[...] + zb0[...] + y20[...]) + b3r[0])
    h1 = jnp.tanh(dinv * (za1[...] + zb1[...] + y21[...]) + b3r[1])
    val = h0 * Wrr[0, 0] + h1 * Wrr[1, 0]
    gid = i * _BT + lax.broadcasted_iota(jnp.int32, (1, _BT), 1)
    masked = jnp.where(gid < _N, val.reshape(1, _BT), 0.0)
    total = jnp.sum(masked)

    @pl.when(i == 0)
    def _init():
      outr[...] = jnp.zeros((1, 1), jnp.float32)

    outr[...] = outr[...] + total

  return pl.pallas_call(
      body,
      grid=(_GRID,),
      in_specs=[_BS] * 7 + [_SM] * 2,
      out_specs=pl.BlockSpec((1, 1), lambda i: (0, 0)),
      out_shape=_F32((1, 1)),
  )(*z2, *y2, dinv, b3, Wr)


def kernel(x, edge_index, W1, b1, W2, b2, W3, b3, Wr, br):
  src = edge_index[0].astype(jnp.int32)
  dst = edge_index[1].astype(jnp.int32)
  pad = _N + (jnp.arange(_EPAD - _E, dtype=jnp.int32) % 32)
  src2 = jnp.concatenate([src, pad]).reshape(_WINDOWS, _LANE)
  dst2 = jnp.concatenate([dst, pad]).reshape(_WINDOWS, _LANE)

  xp = jnp.pad(x.astype(jnp.float32), ((0, _NPAD - _N), (0, 0)))
  x0 = xp[:, 0]
  x1 = xp[:, 1]

  dp0, dp1 = _deg_call(dst2)
  dinv, y00, y01 = _t1_call(x0, x1, dp0, dp1)
  z0 = _agg_call(2, src2, dst2, [y00, y01])
  y1 = _t2_call(z0, [y00, y01], dinv, W1, b1, W2)
  z1 = _agg_call(4, src2, dst2, y1)
  y2 = _t3_call(z1, y1, dinv, b2, W3)
  z2 = _agg_call(2, src2, dst2, y2)
  t4 = _t4_call(z2, y2, dinv, b3, Wr)
  return t4[0, 0] + jnp.float32(_N) * br[0]


# speedup vs baseline: 108.8614x; 1.2273x over previous
import functools

import jax
import jax.numpy as jnp
from jax import lax
from jax.experimental import pallas as pl
from jax.experimental.pallas import tpu as pltpu
from jax.experimental.pallas import tpu_sc as plsc

_N = 100000
_E = 6400000
_NPAD = 100352
_SLICE = _NPAD // 16
_LANE = 128
_WINDOWS = 50176
_EPAD = _WINDOWS * _LANE
_WTILE = _WINDOWS // 32
_KW = 16
_CHUNKS = _WTILE // _KW
_BT = 2048
_GRID = _NPAD // _BT

_MESH = dict(core_axis_name="c", subcore_axis_name="s")


def _deg_call(dst2):
  out_type = [jax.ShapeDtypeStruct((_NPAD,), jnp.float32) for _ in range(2)]
  scratch = [
      pltpu.VMEM_SHARED((_NPAD,), jnp.float32),
      pltpu.VMEM((2, _KW, _LANE), jnp.int32),
      pltpu.VMEM((_LANE,), jnp.float32),
      pltpu.VMEM((_SLICE,), jnp.float32),
      pltpu.SemaphoreType.DMA,
      pltpu.SemaphoreType.DMA,
  ]

  def body(dst_hbm, out0, out1, dsh, dbuf, ones, zbuf, stage_sem, ssem):
    c = lax.axis_index("c")
    s = lax.axis_index("s")
    wid = s * 2 + c
    r0 = s * _SLICE

    z16 = jnp.zeros((16,), jnp.float32)
    o16 = jnp.ones((16,), jnp.float32)

    def zb(i, carry):
      zbuf[pl.ds(i * 16, 16)] = z16
      return carry
    lax.fori_loop(0, _SLICE // 16, zb, 0)
    for i in range(_LANE // 16):
      ones[pl.ds(i * 16, 16)] = o16
    pltpu.sync_copy(zbuf, dsh.at[pl.ds(r0, _SLICE)])
    plsc.subcore_barrier()

    w_base = wid * _WTILE
    pltpu.async_copy(dst_hbm.at[pl.ds(w_base, _KW), :], dbuf.at[0], stage_sem)

    def chunk(t, carry):
      b = lax.rem(t, 2)
      w0 = w_base + t * _KW
      pltpu.make_async_copy(
          dst_hbm.at[pl.ds(w0, _KW), :], dbuf.at[b], stage_sem).wait()

      @pl.when(t < _CHUNKS - 1)
      def _prefetch():
        pltpu.async_copy(
            dst_hbm.at[pl.ds(w0 + _KW, _KW), :], dbuf.at[1 - b], stage_sem)

      sd = []
      for j in range(_KW):
        sd.append(
            pltpu.async_copy(ones, dsh.at[dbuf.at[b, j]], ssem, add=True))
      for d in sd:
        d.wait()
      return carry

    lax.fori_loop(0, _CHUNKS, chunk, 0)
    plsc.subcore_barrier()

    @pl.when(c == 0)
    def _w0():
      pltpu.sync_copy(dsh.at[pl.ds(r0, _SLICE)], out0.at[pl.ds(r0, _SLICE)])

    @pl.when(c == 1)
    def _w1():
      pltpu.sync_copy(dsh.at[pl.ds(r0, _SLICE)], out1.at[pl.ds(r0, _SLICE)])

  fn = pl.kernel(
      body,
      out_type=out_type,
      mesh=plsc.VectorSubcoreMesh(**_MESH),
      scratch_types=scratch,
  )
  return fn(dst2)


def _agg_call(F, src2, dst2, ycols):
  out_type = [jax.ShapeDtypeStruct((_NPAD,), jnp.float32) for _ in range(2 * F)]
  scratch = (
      [pltpu.VMEM_SHARED((_NPAD,), jnp.float32) for _ in range(2 * F)]
      + [pltpu.VMEM((3, _KW, _LANE), jnp.int32) for _ in range(2)]
      + [pltpu.VMEM((2, _KW, _LANE), jnp.float32) for _ in range(F)]
      + [pltpu.VMEM((_SLICE,), jnp.float32)]
      + [pltpu.SemaphoreType.DMA for _ in range(3)]
  )

  def body(*refs):
    src_hbm, dst_hbm = refs[0], refs[1]
    yin = refs[2:2 + F]
    zout = refs[2 + F:2 + 3 * F]
    p = 2 + 3 * F
    ysh = refs[p:p + F]; p += F
    zsh = refs[p:p + F]; p += F
    sbuf, dbuf = refs[p], refs[p + 1]; p += 2
    grows = refs[p:p + F]; p += F
    zbuf = refs[p]; p += 1
    stage_sem, gsem, ssem = refs[p], refs[p + 1], refs[p + 2]

    c = lax.axis_index("c")
    s = lax.axis_index("s")
    wid = s * 2 + c
    r0 = s * _SLICE

    z16 = jnp.zeros((16,), jnp.float32)

    def zb(i, carry):
      zbuf[pl.ds(i * 16, 16)] = z16
      return carry
    lax.fori_loop(0, _SLICE // 16, zb, 0)

    for f in range(F):
      pltpu.sync_copy(yin[f].at[pl.ds(r0, _SLICE)],
                      ysh[f].at[pl.ds(r0, _SLICE)])
      pltpu.sync_copy(zbuf, zsh[f].at[pl.ds(r0, _SLICE)])
    plsc.subcore_barrier()

    w_base = wid * _WTILE
    pltpu.async_copy(src_hbm.at[pl.ds(w_base, _KW), :], sbuf.at[0], stage_sem)
    pltpu.async_copy(dst_hbm.at[pl.ds(w_base, _KW), :], dbuf.at[0], stage_sem)

    def chunk(t, carry):
      b2 = lax.rem(t, 2)
      b3 = lax.rem(t, 3)
      n3 = lax.rem(t + 1, 3)
      p3 = lax.rem(t + 2, 3)
      w0 = w_base + t * _KW
      pltpu.make_async_copy(
          src_hbm.at[pl.ds(w0, _KW), :], sbuf.at[b3], stage_sem).wait()
      pltpu.make_async_copy(
          dst_hbm.at[pl.ds(w0, _KW), :], dbuf.at[b3], stage_sem).wait()

      @pl.when(t < _CHUNKS - 1)
      def _prefetch():
        pltpu.async_copy(
            src_hbm.at[pl.ds(w0 + _KW, _KW), :], sbuf.at[n3], stage_sem)
        pltpu.async_copy(
            dst_hbm.at[pl.ds(w0 + _KW, _KW), :], dbuf.at[n3], stage_sem)

      gd = []
      for j in range(_KW):
        for f in range(F):
          gd.append(pltpu.async_copy(
              ysh[f].at[sbuf.at[b3, j]], grows[f].at[b2, j], gsem))

      @pl.when(t > 0)
      def _drain_prev():
        for j in range(_KW):
          for f in range(F):
            pltpu.make_async_copy(
                grows[f].at[1 - b2, j],
                zsh[f].at[dbuf.at[p3, j]], ssem).wait()

      for d in gd:
        d.wait()
      for j in range(_KW):
        for f in range(F):
          pltpu.async_copy(
              grows[f].at[b2, j], zsh[f].at[dbuf.at[b3, j]], ssem, add=True)
      return carry

    lax.fori_loop(0, _CHUNKS, chunk, 0)

    lb2 = (_CHUNKS - 1) % 2
    lb3 = (_CHUNKS - 1) % 3
    for j in range(_KW):
      for f in range(F):
        pltpu.make_async_copy(
            grows[f].at[lb2, j], zsh[f].at[dbuf.at[lb3, j]], ssem).wait()
    plsc.subcore_barrier()

    for f in range(F):
      @pl.when(c == 0)
      def _w0(f=f):
        pltpu.sync_copy(zsh[f].at[pl.ds(r0, _SLICE)],
                        zout[f].at[pl.ds(r0, _SLICE)])

      @pl.when(c == 1)
      def _w1(f=f):
        pltpu.sync_copy(zsh[f].at[pl.ds(r0, _SLICE)],
                        zout[F + f].at[pl.ds(r0, _SLICE)])

  fn = pl.kernel(
      body,
      out_type=out_type,
      mesh=plsc.VectorSubcoreMesh(**_MESH),
      scratch_types=scratch,
  )
  return fn(src2, dst2, *ycols)


_BS = pl.BlockSpec((_BT,), lambda i: (i,))
_SM = pl.BlockSpec(memory_space=pltpu.SMEM)
_F32 = functools.partial(jax.ShapeDtypeStruct, dtype=jnp.float32)


def _t1_call(x0, x1, dp0, dp1):
  def body(x0r, x1r, d0r, d1r, dinvr, y00r, y01r):
    dinv = lax.rsqrt(d0r[...] + d1r[...] + 1.0)
    dinvr[...] = dinv
    y00r[...] = x0r[...] * dinv
    y01r[...] = x1r[...] * dinv

  return pl.pallas_call(
      body,
      grid=(_GRID,),
      in_specs=[_BS] * 4,
      out_specs=[_BS] * 3,
      out_shape=[_F32((_NPAD,))] * 3,
  )(x0, x1, dp0, dp1)


def _t2_call(z0, y0, dinv, W1, b1, W2):
  def body(za0, za1, zb0, zb1, y00, y01, dinvr, W1r, b1r, W2r, *outs):
    dinv = dinvr[...]
    ax = [dinv * (za0[...] + zb0[...] + y00[...]),
          dinv * (za1[...] + zb1[...] + y01[...])]
    h = []
    for j in range(4):
      g = ax[0] * W1r[0, j] + ax[1] * W1r[1, j] + b1r[j]
      h.append(jnp.tanh(g))
    for j in range(4):
      acc = h[0] * W2r[0, j]
      for i in range(1, 4):
        acc = acc + h[i] * W2r[i, j]
      outs[j][...] = dinv * acc

  return pl.pallas_call(
      body,
      grid=(_GRID,),
      in_specs=[_BS] * 7 + [_SM] * 3,
      out_specs=[_BS] * 4,
      out_shape=[_F32((_NPAD,))] * 4,
  )(z0[0], z0[1], z0[2], z0[3], y0[0], y0[1], dinv, W1, b1, W2)


def _t3_call(z1, y1, dinv, b2, W3):
  def body(za0, za1, za2, za3, zb0, zb1, zb2, zb3,
           y10, y11, y12, y13, dinvr, b2r, W3r, o0, o1):
    dinv = dinvr[...]
    za = [za0, za1, za2, za3]
    zb = [zb0, zb1, zb2, zb3]
    yy = [y10, y11, y12, y13]
    h = []
    for j in range(4):
      h.append(jnp.tanh(dinv * (za[j][...] + zb[j][...] + yy[j][...])
                        + b2r[j]))
    for k, o in enumerate((o0, o1)):
      acc = h[0] * W3r[0, k]
      for i in range(1, 4):
        acc = acc + h[i] * W3r[i, k]
      o[...] = dinv * acc

  return pl.pallas_call(
      body,
      grid=(_GRID,),
      in_specs=[_BS] * 13 + [_SM] * 2,
      out_specs=[_BS] * 2,
      out_shape=[_F32((_NPAD,))] * 2,
  )(*z1, *y1, dinv, b2, W3)


def _t4_call(z2, y2, dinv, b3, Wr):
  def body(za0, za1, zb0, zb1, y20, y21, dinvr, b3r, Wrr, outr):
    i = pl.program_id(0)
    dinv = dinvr[...]
    h0 = jnp.tanh(dinv * (za0[...] + zb0[...] + y20[...]) + b3r[0])
    h1 = jnp.tanh(dinv * (za1[...] + zb1[...] + y21[...]) + b3r[1])
    val = h0 * Wrr[0, 0] + h1 * Wrr[1, 0]
    gid = i * _BT + lax.broadcasted_iota(jnp.int32, (1, _BT), 1)
    masked = jnp.where(gid < _N, val.reshape(1, _BT), 0.0)
    total = jnp.sum(masked)

    @pl.when(i == 0)
    def _init():
      outr[...] = jnp.zeros((1, 1), jnp.float32)

    outr[...] = outr[...] + total

  return pl.pallas_call(
      body,
      grid=(_GRID,),
      in_specs=[_BS] * 7 + [_SM] * 2,
      out_specs=pl.BlockSpec((1, 1), lambda i: (0, 0)),
      out_shape=_F32((1, 1)),
  )(*z2, *y2, dinv, b3, Wr)


def kernel(x, edge_index, W1, b1, W2, b2, W3, b3, Wr, br):
  src = edge_index[0].astype(jnp.int32)
  dst = edge_index[1].astype(jnp.int32)
  pad = _N + (jnp.arange(_EPAD - _E, dtype=jnp.int32) % 32)
  src2 = jnp.concatenate([src, pad]).reshape(_WINDOWS, _LANE)
  dst2 = jnp.concatenate([dst, pad]).reshape(_WINDOWS, _LANE)

  xp = jnp.pad(x.astype(jnp.float32), ((0, _NPAD - _N), (0, 0)))
  x0 = xp[:, 0]
  x1 = xp[:, 1]

  dp0, dp1 = _deg_call(dst2)
  dinv, y00, y01 = _t1_call(x0, x1, dp0, dp1)
  z0 = _agg_call(2, src2, dst2, [y00, y01])
  y1 = _t2_call(z0, [y00, y01], dinv, W1, b1, W2)
  z1 = _agg_call(4, src2, dst2, y1)
  y2 = _t3_call(z1, y1, dinv, b2, W3)
  z2 = _agg_call(2, src2, dst2, y2)
  t4 = _t4_call(z2, y2, dinv, b3, Wr)
  return t4[0, 0] + jnp.float32(_N) * br[0]
